# Initial kernel scaffold; baseline (speedup 1.0000x reference)
import jax
import jax.numpy as jnp
from jax.experimental import pallas as pl


def kernel(value, key, query, mask_ori, Wv, Wk, Wq, Wf, bf):
    raise NotImplementedError("write your pallas kernel here")


# speedup vs baseline: 22397.3221x; 22397.3221x over previous
import functools

import jax
import jax.numpy as jnp
from jax import lax
from jax.experimental import pallas as pl
from jax.experimental.pallas import tpu as pltpu
from jax.experimental.pallas import tpu_sc as plsc

MASK_ID = 600
KP = 640
WPAD = 64
NEG = -1e20

def _dot(a, b, dims):
    return lax.dot_general(a, b, (dims, ((), ())),
                           preferred_element_type=jnp.float32)


def _score_body(q_ref, k_ref, wq_ref, wk_ref, s_ref):
    qp = _dot(q_ref[0], wq_ref[...], ((1,), (1,)))
    kp = _dot(k_ref[0], wk_ref[...], ((1,), (1,)))
    s_ref[0] = _dot(qp, kp, ((1,), (1,)))


def _out_body(a_ref, v_ref, wv_ref, wf_ref, bf_ref, o_ref):
    vp = _dot(v_ref[0], wv_ref[...], ((1,), (1,)))
    x = _dot(a_ref[0], vp, ((1,), (0,)))
    o_ref[0] = _dot(x, wf_ref[...], ((1,), (1,))) + bf_ref[...]


def _make_sc_attend(n_rows, w):
    info = plsc.get_sparse_core_info()
    nc, ns = info.num_cores, info.num_subcores
    nw = nc * ns
    rows_per = n_rows // nw
    n_wv = WPAD // 16

    mesh = plsc.VectorSubcoreMesh(core_axis_name="c", subcore_axis_name="s")

    @functools.partial(
        pl.kernel,
        mesh=mesh,
        compiler_params=pltpu.CompilerParams(needs_layout_passes=False),
        out_type=[
            jax.ShapeDtypeStruct((n_rows * WPAD,), jnp.float32),
            jax.ShapeDtypeStruct((n_rows * KP,), jnp.float32),
        ],
        scratch_types=[
            pltpu.VMEM((rows_per * KP,), jnp.float32),
            pltpu.VMEM((rows_per * WPAD,), jnp.int32),
            pltpu.VMEM((rows_per * WPAD,), jnp.float32),
            pltpu.VMEM((rows_per * KP,), jnp.float32),
        ],
    )
    def sc_attend(s_hbm, idx_hbm, attn_hbm, a_hbm, s_v, idx_v, attn_v, a_v):
        wid = lax.axis_index("s") * nc + lax.axis_index("c")
        rbase = wid * rows_per
        pltpu.sync_copy(s_hbm.at[pl.ds(rbase * KP, rows_per * KP)], s_v)
        pltpu.sync_copy(idx_hbm.at[pl.ds(rbase * WPAD, rows_per * WPAD)], idx_v)

        lane = lax.iota(jnp.int32, 16)
        zero16 = jnp.zeros((16,), jnp.float32)

        def row_body(r, carry):
            rb = r * KP

            def zbody(j, c):
                a_v[pl.ds(rb + j * 16, 16)] = zero16
                return c

            lax.fori_loop(0, KP // 16, zbody, 0)

            iws, es = [], []
            for v in range(n_wv):
                iw = idx_v[pl.ds(r * WPAD + v * 16, 16)]
                e = plsc.load_gather(s_v, [iw + rb])
                bad = iw == MASK_ID
                if (v + 1) * 16 > w:
                    bad = bad | (lane >= (w - v * 16))
                iws.append(iw)
                es.append(jnp.where(bad, NEG, e))

            m01 = jnp.maximum(es[0], es[1])
            m23 = jnp.maximum(es[2], es[3])
            m = jnp.max(jnp.maximum(m01, m23))
            ps = [jnp.exp(e - m) for e in es]
            for v in range(n_wv):
                if (v + 1) * 16 > w:
                    ps[v] = jnp.where(lane < (w - v * 16), ps[v], 0.0)
            tot = jnp.sum((ps[0] + ps[1]) + (ps[2] + ps[3]))
            inv = jnp.ones((16,), jnp.float32) / tot
            for v in range(n_wv):
                aw = ps[v] * inv
                attn_v[pl.ds(r * WPAD + v * 16, 16)] = aw
                if (v + 1) * 16 > w:
                    plsc.addupdate_scatter(a_v, [iws[v] + rb], aw,
                                           mask=lane < (w - v * 16))
                else:
                    plsc.addupdate_scatter(a_v, [iws[v] + rb], aw)
            return carry

        lax.fori_loop(0, rows_per, row_body, 0)
        pltpu.sync_copy(attn_v, attn_hbm.at[pl.ds(rbase * WPAD, rows_per * WPAD)])
        pltpu.sync_copy(a_v, a_hbm.at[pl.ds(rbase * KP, rows_per * KP)])

    return sc_attend


def kernel(value, key, query, mask_ori, Wv, Wk, Wq, Wf, bf):
    n, l, hd = query.shape
    vd = value.shape[2]
    w = mask_ori.shape[2]
    nl = n * l

    k_tab = key[:, :KP]
    v_tab = value[:, :KP]

    scores = pl.pallas_call(
        _score_body,
        grid=(n,),
        in_specs=[
            pl.BlockSpec((1, l, hd), lambda i: (i, 0, 0)),
            pl.BlockSpec((1, KP, hd), lambda i: (i, 0, 0)),
            pl.BlockSpec((hd, hd), lambda i: (0, 0)),
            pl.BlockSpec((hd, hd), lambda i: (0, 0)),
        ],
        out_specs=pl.BlockSpec((1, l, KP), lambda i: (i, 0, 0)),
        out_shape=jax.ShapeDtypeStruct((n, l, KP), jnp.float32),
    )(query, k_tab, Wq, Wk)

    idx_pad = jnp.pad(mask_ori.reshape(nl, w), ((0, 0), (0, WPAD - w)))

    attn_flat, a_flat = _make_sc_attend(nl, w)(
        scores.reshape(nl * KP), idx_pad.reshape(nl * WPAD))

    attn = attn_flat.reshape(nl, WPAD)[:, :w].reshape(n, l, w)
    a_dense = a_flat.reshape(n, l, KP)

    out = pl.pallas_call(
        _out_body,
        grid=(n,),
        in_specs=[
            pl.BlockSpec((1, l, KP), lambda i: (i, 0, 0)),
            pl.BlockSpec((1, KP, vd), lambda i: (i, 0, 0)),
            pl.BlockSpec((vd, vd), lambda i: (0, 0)),
            pl.BlockSpec((vd, vd), lambda i: (0, 0)),
            pl.BlockSpec((1, vd), lambda i: (0, 0)),
        ],
        out_specs=pl.BlockSpec((1, l, vd), lambda i: (i, 0, 0)),
        out_shape=jax.ShapeDtypeStruct((n, l, vd), jnp.float32),
    )(a_dense, v_tab, Wv, Wf, bf.reshape(1, vd))

    return out, attn


# speedup vs baseline: 22517.0636x; 1.0053x over previous
import functools

import jax
import jax.numpy as jnp
from jax import lax
from jax.experimental import pallas as pl
from jax.experimental.pallas import tpu as pltpu
from jax.experimental.pallas import tpu_sc as plsc

MASK_ID = 600
KP = 640
WPAD = 64
NEG = -1e20

def _dot(a, b, dims):
    return lax.dot_general(a, b, (dims, ((), ())),
                           preferred_element_type=jnp.float32)


def _score_body(q_ref, k_ref, wq_ref, wk_ref, s_ref):
    qp = _dot(q_ref[0], wq_ref[...], ((1,), (1,)))
    kp = _dot(k_ref[0], wk_ref[...], ((1,), (1,)))
    s_ref[0] = _dot(qp, kp, ((1,), (1,)))


def _out_body(a_ref, v_ref, wv_ref, wf_ref, bf_ref, o_ref):
    vp = _dot(v_ref[0], wv_ref[...], ((1,), (1,)))
    x = _dot(a_ref[0], vp, ((1,), (0,)))
    o_ref[0] = _dot(x, wf_ref[...], ((1,), (1,))) + bf_ref[...]


def _make_sc_attend(n_rows, w):
    info = plsc.get_sparse_core_info()
    nc, ns = info.num_cores, info.num_subcores
    nw = nc * ns
    rows_per = n_rows // nw

    mesh = plsc.VectorSubcoreMesh(core_axis_name="c", subcore_axis_name="s")

    @functools.partial(
        pl.kernel,
        mesh=mesh,
        compiler_params=pltpu.CompilerParams(needs_layout_passes=False),
        out_type=[
            jax.ShapeDtypeStruct((n_rows * WPAD,), jnp.float32),
            jax.ShapeDtypeStruct((n_rows * KP,), jnp.float32),
        ],
        scratch_types=[
            pltpu.VMEM((rows_per * KP,), jnp.float32),
            pltpu.VMEM((rows_per * WPAD,), jnp.int32),
            pltpu.VMEM((rows_per * WPAD,), jnp.float32),
            pltpu.VMEM((rows_per * KP,), jnp.float32),
            pltpu.VMEM((w * 16,), jnp.float32),
        ],
    )
    def sc_attend(s_hbm, idx_hbm, attn_hbm, a_hbm, s_v, idx_v, attn_v, a_v, e_buf):
        wid = lax.axis_index("s") * nc + lax.axis_index("c")
        rbase = wid * rows_per
        pltpu.sync_copy(s_hbm.at[pl.ds(rbase * KP, rows_per * KP)], s_v)
        pltpu.sync_copy(idx_hbm.at[pl.ds(rbase * WPAD, rows_per * WPAD)], idx_v)

        lane = lax.iota(jnp.int32, 16)
        zero16 = jnp.zeros((16,), jnp.float32)
        neg16 = jnp.full((16,), NEG, jnp.float32)
        one16 = jnp.ones((16,), jnp.float32)

        def zbody(j, c):
            for u in range(8):
                a_v[pl.ds(j * 128 + u * 16, 16)] = zero16
            return c

        lax.fori_loop(0, rows_per * KP // 128, zbody, 0)

        for g in range(rows_per // 16):
            row_off = (g * 16 + lane) * KP
            idx_off = (g * 16 + lane) * WPAD

            def pass1(wi, carry):
                mx, sm = carry
                iw = plsc.load_gather(idx_v, [idx_off + wi])
                e = plsc.load_gather(s_v, [row_off + iw])
                e = jnp.where(iw == MASK_ID, neg16, e)
                e_buf[pl.ds(wi * 16, 16)] = e
                nmx = jnp.maximum(mx, e)
                sm = sm * jnp.exp(mx - nmx) + jnp.exp(e - nmx)
                return nmx, sm

            mx, sm = lax.fori_loop(0, w, pass1, (neg16, zero16))
            inv = one16 / sm

            def pass2(wi, carry):
                e = e_buf[pl.ds(wi * 16, 16)]
                aw = jnp.exp(e - mx) * inv
                iw = plsc.load_gather(idx_v, [idx_off + wi])
                plsc.store_scatter(attn_v, [idx_off + wi], aw)
                plsc.addupdate_scatter(a_v, [row_off + iw], aw)
                return carry

            lax.fori_loop(0, w, pass2, 0)

        pltpu.sync_copy(attn_v, attn_hbm.at[pl.ds(rbase * WPAD, rows_per * WPAD)])
        pltpu.sync_copy(a_v, a_hbm.at[pl.ds(rbase * KP, rows_per * KP)])

    return sc_attend


def kernel(value, key, query, mask_ori, Wv, Wk, Wq, Wf, bf):
    n, l, hd = query.shape
    vd = value.shape[2]
    w = mask_ori.shape[2]
    nl = n * l

    k_tab = key[:, :KP]
    v_tab = value[:, :KP]

    scores = pl.pallas_call(
        _score_body,
        grid=(n,),
        in_specs=[
            pl.BlockSpec((1, l, hd), lambda i: (i, 0, 0)),
            pl.BlockSpec((1, KP, hd), lambda i: (i, 0, 0)),
            pl.BlockSpec((hd, hd), lambda i: (0, 0)),
            pl.BlockSpec((hd, hd), lambda i: (0, 0)),
        ],
        out_specs=pl.BlockSpec((1, l, KP), lambda i: (i, 0, 0)),
        out_shape=jax.ShapeDtypeStruct((n, l, KP), jnp.float32),
    )(query, k_tab, Wq, Wk)

    idx_pad = jnp.pad(mask_ori.reshape(nl, w), ((0, 0), (0, WPAD - w)))

    attn_flat, a_flat = _make_sc_attend(nl, w)(
        scores.reshape(nl * KP), idx_pad.reshape(nl * WPAD))

    attn = attn_flat.reshape(nl, WPAD)[:, :w].reshape(n, l, w)
    a_dense = a_flat.reshape(n, l, KP)

    out = pl.pallas_call(
        _out_body,
        grid=(n,),
        in_specs=[
            pl.BlockSpec((1, l, KP), lambda i: (i, 0, 0)),
            pl.BlockSpec((1, KP, vd), lambda i: (i, 0, 0)),
            pl.BlockSpec((vd, vd), lambda i: (0, 0)),
            pl.BlockSpec((vd, vd), lambda i: (0, 0)),
            pl.BlockSpec((1, vd), lambda i: (0, 0)),
        ],
        out_specs=pl.BlockSpec((1, l, vd), lambda i: (i, 0, 0)),
        out_shape=jax.ShapeDtypeStruct((n, l, vd), jnp.float32),
    )(a_dense, v_tab, Wv, Wf, bf.reshape(1, vd))

    return out, attn


# speedup vs baseline: 28716.5797x; 1.2753x over previous
import functools

import jax
import jax.numpy as jnp
from jax import lax
from jax.experimental import pallas as pl
from jax.experimental.pallas import tpu as pltpu
from jax.experimental.pallas import tpu_sc as plsc

MASK_ID = 600
KP = 640
WPAD = 64
NEG = -1e20


def _dot(a, b, dims):
    return lax.dot_general(a, b, (dims, ((), ())),
                           preferred_element_type=jnp.float32)


def _score_body(q_ref, k_ref, wq_ref, wk_ref, s_ref):
    qp = _dot(q_ref[0], wq_ref[...], ((1,), (1,)))
    kp = _dot(k_ref[0], wk_ref[...], ((1,), (1,)))
    s_ref[...] = _dot(qp, kp, ((1,), (1,)))


def _out_body(a_ref, v_ref, wv_ref, wf_ref, bf_ref, o_ref):
    vp = _dot(v_ref[0], wv_ref[...], ((1,), (1,)))
    x = _dot(a_ref[...], vp, ((1,), (0,)))
    o_ref[0] = _dot(x, wf_ref[...], ((1,), (1,))) + bf_ref[...]


def _make_sc_attend(n_rows, w):
    info = plsc.get_sparse_core_info()
    nc, ns = info.num_cores, info.num_subcores
    nw = nc * ns
    rows_per = n_rows // nw

    mesh = plsc.VectorSubcoreMesh(core_axis_name="c", subcore_axis_name="s")

    @functools.partial(
        pl.kernel,
        mesh=mesh,
        compiler_params=pltpu.CompilerParams(needs_layout_passes=False),
        out_type=[
            jax.ShapeDtypeStruct((n_rows * WPAD,), jnp.float32),
            jax.ShapeDtypeStruct((n_rows, KP), jnp.float32),
        ],
        scratch_types=[
            pltpu.VMEM((rows_per, KP), jnp.float32),
            pltpu.VMEM((rows_per * WPAD,), jnp.int32),
            pltpu.VMEM((rows_per * WPAD,), jnp.float32),
            pltpu.VMEM((rows_per, KP), jnp.float32),
            pltpu.VMEM((w * 16,), jnp.float32),
        ],
    )
    def sc_attend(s_hbm, idx_hbm, attn_hbm, a_hbm, s_v, idx_v, attn_v, a_v, e_buf):
        wid = lax.axis_index("s") * nc + lax.axis_index("c")
        rbase = wid * rows_per
        pltpu.sync_copy(s_hbm.at[pl.ds(rbase, rows_per)], s_v)
        pltpu.sync_copy(idx_hbm.at[pl.ds(rbase * WPAD, rows_per * WPAD)], idx_v)

        lane = lax.iota(jnp.int32, 16)
        zero16 = jnp.zeros((16,), jnp.float32)
        neg16 = jnp.full((16,), NEG, jnp.float32)
        one16 = jnp.ones((16,), jnp.float32)

        def zbody(r, c):
            for u in range(KP // 16):
                a_v[r, pl.ds(u * 16, 16)] = zero16
            return c

        lax.fori_loop(0, rows_per, zbody, 0)

        for g in range(rows_per // 16):
            row16 = g * 16 + lane
            idx_off = row16 * WPAD

            def pass1(wi, carry):
                mx, sm = carry
                iw = plsc.load_gather(idx_v, [idx_off + wi])
                e = plsc.load_gather(s_v, [row16, iw])
                e = jnp.where(iw == MASK_ID, neg16, e)
                e_buf[pl.ds(wi * 16, 16)] = e
                nmx = jnp.maximum(mx, e)
                sm = sm * jnp.exp(mx - nmx) + jnp.exp(e - nmx)
                return nmx, sm

            mx, sm = lax.fori_loop(0, w, pass1, (neg16, zero16))
            inv = one16 / sm

            def pass2(wi, carry):
                e = e_buf[pl.ds(wi * 16, 16)]
                aw = jnp.exp(e - mx) * inv
                iw = plsc.load_gather(idx_v, [idx_off + wi])
                plsc.store_scatter(attn_v, [idx_off + wi], aw)
                plsc.addupdate_scatter(a_v, [row16, iw], aw)
                return carry

            lax.fori_loop(0, w, pass2, 0)

        pltpu.sync_copy(attn_v, attn_hbm.at[pl.ds(rbase * WPAD, rows_per * WPAD)])
        pltpu.sync_copy(a_v, a_hbm.at[pl.ds(rbase, rows_per)])

    return sc_attend


def kernel(value, key, query, mask_ori, Wv, Wk, Wq, Wf, bf):
    n, l, hd = query.shape
    vd = value.shape[2]
    w = mask_ori.shape[2]
    nl = n * l

    scores = pl.pallas_call(
        _score_body,
        grid=(n,),
        in_specs=[
            pl.BlockSpec((1, l, hd), lambda i: (i, 0, 0)),
            pl.BlockSpec((1, KP, hd), lambda i: (i, 0, 0)),
            pl.BlockSpec((hd, hd), lambda i: (0, 0)),
            pl.BlockSpec((hd, hd), lambda i: (0, 0)),
        ],
        out_specs=pl.BlockSpec((l, KP), lambda i: (i, 0)),
        out_shape=jax.ShapeDtypeStruct((nl, KP), jnp.float32),
    )(query, key, Wq, Wk)

    idx_pad = jnp.pad(mask_ori.reshape(nl, w), ((0, 0), (0, WPAD - w)))

    attn_flat, a_dense = _make_sc_attend(nl, w)(
        scores, idx_pad.reshape(nl * WPAD))

    attn = attn_flat.reshape(nl, WPAD)[:, :w].reshape(n, l, w)

    out = pl.pallas_call(
        _out_body,
        grid=(n,),
        in_specs=[
            pl.BlockSpec((l, KP), lambda i: (i, 0)),
            pl.BlockSpec((1, KP, vd), lambda i: (i, 0, 0)),
            pl.BlockSpec((vd, vd), lambda i: (0, 0)),
            pl.BlockSpec((vd, vd), lambda i: (0, 0)),
            pl.BlockSpec((1, vd), lambda i: (0, 0)),
        ],
        out_specs=pl.BlockSpec((1, l, vd), lambda i: (i, 0, 0)),
        out_shape=jax.ShapeDtypeStruct((n, l, vd), jnp.float32),
    )(a_dense, value, Wv, Wf, bf.reshape(1, vd))

    return out, attn


# speedup vs baseline: 30180.8872x; 1.0510x over previous
import functools

import jax
import jax.numpy as jnp
from jax import lax
from jax.experimental import pallas as pl
from jax.experimental.pallas import tpu as pltpu
from jax.experimental.pallas import tpu_sc as plsc

MASK_ID = 600
KP = 640
WPAD = 64
NEG = -1e20


def _dot(a, b, dims):
    return lax.dot_general(a, b, (dims, ((), ())),
                           preferred_element_type=jnp.float32)


def _score_body(q_ref, k_ref, wq_ref, wk_ref, s_ref):
    qp = _dot(q_ref[0], wq_ref[...], ((1,), (1,)))
    kp = _dot(k_ref[0], wk_ref[...], ((1,), (1,)))
    s_ref[...] = _dot(qp, kp, ((1,), (1,)))


def _out_body(a_ref, v_ref, wv_ref, wf_ref, bf_ref, o_ref):
    vp = _dot(v_ref[0], wv_ref[...], ((1,), (1,)))
    x = _dot(a_ref[...], vp, ((1,), (0,)))
    o_ref[0] = _dot(x, wf_ref[...], ((1,), (1,))) + bf_ref[...]


def _make_sc_attend(n_rows, w):
    info = plsc.get_sparse_core_info()
    nc, ns = info.num_cores, info.num_subcores
    nw = nc * ns
    rows_per = n_rows // nw

    mesh = plsc.VectorSubcoreMesh(core_axis_name="c", subcore_axis_name="s")

    n_g = rows_per // 16

    scratch = [
        pltpu.VMEM((rows_per, KP), jnp.float32),
        pltpu.VMEM((rows_per * WPAD,), jnp.int32),
        pltpu.VMEM((rows_per * WPAD,), jnp.float32),
        pltpu.VMEM((rows_per, KP), jnp.float32),
        pltpu.VMEM((w * 16,), jnp.float32),
    ]
    scratch += [pltpu.SemaphoreType.DMA] * (3 * n_g)

    @functools.partial(
        pl.kernel,
        mesh=mesh,
        compiler_params=pltpu.CompilerParams(needs_layout_passes=False),
        out_type=[
            jax.ShapeDtypeStruct((n_rows * WPAD,), jnp.float32),
            jax.ShapeDtypeStruct((n_rows, KP), jnp.float32),
        ],
        scratch_types=scratch,
    )
    def sc_attend(s_hbm, idx_hbm, attn_hbm, a_hbm, s_v, idx_v, attn_v, a_v,
                  e_buf, *sems):
        s_sems, a_sems, at_sems = sems[:n_g], sems[n_g:2 * n_g], sems[2 * n_g:]
        wid = lax.axis_index("s") * nc + lax.axis_index("c")
        rbase = wid * rows_per

        s_cp = [None] * n_g
        s_cp[0] = pltpu.async_copy(s_hbm.at[pl.ds(rbase, 16)],
                                   s_v.at[pl.ds(0, 16)], s_sems[0])
        pltpu.sync_copy(idx_hbm.at[pl.ds(rbase * WPAD, rows_per * WPAD)], idx_v)

        lane = lax.iota(jnp.int32, 16)
        zero16 = jnp.zeros((16,), jnp.float32)
        neg16 = jnp.full((16,), NEG, jnp.float32)
        one16 = jnp.ones((16,), jnp.float32)

        out_cps = []
        for g in range(n_g):
            if g + 1 < n_g:
                s_cp[g + 1] = pltpu.async_copy(
                    s_hbm.at[pl.ds(rbase + (g + 1) * 16, 16)],
                    s_v.at[pl.ds((g + 1) * 16, 16)], s_sems[g + 1])

            def zbody(r, c, g=g):
                for u in range(KP // 16):
                    a_v[g * 16 + r, pl.ds(u * 16, 16)] = zero16
                return c

            lax.fori_loop(0, 16, zbody, 0)
            s_cp[g].wait()

            row16 = g * 16 + lane
            idx_off = row16 * WPAD

            def pass1(wi, carry):
                mx, sm = carry
                iw = plsc.load_gather(idx_v, [idx_off + wi])
                e = plsc.load_gather(s_v, [row16, iw])
                e = jnp.where(iw == MASK_ID, neg16, e)
                e_buf[pl.ds(wi * 16, 16)] = e
                nmx = jnp.maximum(mx, e)
                sm = sm * jnp.exp(mx - nmx) + jnp.exp(e - nmx)
                return nmx, sm

            mx, sm = lax.fori_loop(0, w, pass1, (neg16, zero16))
            inv = one16 / sm

            def pass2(wi, carry):
                e = e_buf[pl.ds(wi * 16, 16)]
                aw = jnp.exp(e - mx) * inv
                iw = plsc.load_gather(idx_v, [idx_off + wi])
                plsc.store_scatter(attn_v, [idx_off + wi], aw)
                plsc.addupdate_scatter(a_v, [row16, iw], aw)
                return carry

            lax.fori_loop(0, w, pass2, 0)

            out_cps.append(pltpu.async_copy(
                a_v.at[pl.ds(g * 16, 16)],
                a_hbm.at[pl.ds(rbase + g * 16, 16)], a_sems[g]))
            out_cps.append(pltpu.async_copy(
                attn_v.at[pl.ds(g * 16 * WPAD, 16 * WPAD)],
                attn_hbm.at[pl.ds((rbase + g * 16) * WPAD, 16 * WPAD)],
                at_sems[g]))

        for cp in out_cps:
            cp.wait()

    return sc_attend


def kernel(value, key, query, mask_ori, Wv, Wk, Wq, Wf, bf):
    n, l, hd = query.shape
    vd = value.shape[2]
    w = mask_ori.shape[2]
    nl = n * l

    scores = pl.pallas_call(
        _score_body,
        grid=(n,),
        in_specs=[
            pl.BlockSpec((1, l, hd), lambda i: (i, 0, 0)),
            pl.BlockSpec((1, KP, hd), lambda i: (i, 0, 0)),
            pl.BlockSpec((hd, hd), lambda i: (0, 0)),
            pl.BlockSpec((hd, hd), lambda i: (0, 0)),
        ],
        out_specs=pl.BlockSpec((l, KP), lambda i: (i, 0)),
        out_shape=jax.ShapeDtypeStruct((nl, KP), jnp.float32),
    )(query, key, Wq, Wk)

    idx_pad = jnp.pad(mask_ori.reshape(nl, w), ((0, 0), (0, WPAD - w)))

    attn_flat, a_dense = _make_sc_attend(nl, w)(
        scores, idx_pad.reshape(nl * WPAD))

    attn = attn_flat.reshape(nl, WPAD)[:, :w].reshape(n, l, w)

    out = pl.pallas_call(
        _out_body,
        grid=(n,),
        in_specs=[
            pl.BlockSpec((l, KP), lambda i: (i, 0)),
            pl.BlockSpec((1, KP, vd), lambda i: (i, 0, 0)),
            pl.BlockSpec((vd, vd), lambda i: (0, 0)),
            pl.BlockSpec((vd, vd), lambda i: (0, 0)),
            pl.BlockSpec((1, vd), lambda i: (0, 0)),
        ],
        out_specs=pl.BlockSpec((1, l, vd), lambda i: (i, 0, 0)),
        out_shape=jax.ShapeDtypeStruct((n, l, vd), jnp.float32),
    )(a_dense, value, Wv, Wf, bf.reshape(1, vd))

    return out, attn


# speedup vs baseline: 31599.7849x; 1.0470x over previous
import functools

import jax
import jax.numpy as jnp
from jax import lax
from jax.experimental import pallas as pl
from jax.experimental.pallas import tpu as pltpu
from jax.experimental.pallas import tpu_sc as plsc

MASK_ID = 600
KP = 640
WPAD = 64
NEG = -1e20


def _dot(a, b, dims):
    return lax.dot_general(a, b, (dims, ((), ())),
                           preferred_element_type=jnp.float32)


def _score_body(q_ref, k_ref, wq_ref, wk_ref, s_ref):
    qp = _dot(q_ref[0], wq_ref[...], ((1,), (1,)))
    kp = _dot(k_ref[0], wk_ref[...], ((1,), (1,)))
    s_ref[...] = _dot(qp, kp, ((1,), (1,)))


def _out_body(a_ref, v_ref, wv_ref, wf_ref, bf_ref, o_ref):
    vp = _dot(v_ref[0], wv_ref[...], ((1,), (1,)))
    x = _dot(a_ref[...], vp, ((1,), (0,)))
    o_ref[0] = _dot(x, wf_ref[...], ((1,), (1,))) + bf_ref[...]


def _make_sc_attend(n_rows, w):
    info = plsc.get_sparse_core_info()
    nc, ns = info.num_cores, info.num_subcores
    nw = nc * ns
    rows_per = n_rows // nw

    mesh = plsc.VectorSubcoreMesh(core_axis_name="c", subcore_axis_name="s")

    n_g = rows_per // 16
    n_t = n_g // 2

    scratch = [
        pltpu.VMEM((rows_per, KP), jnp.float32),
        pltpu.VMEM((rows_per * w,), jnp.int32),
        pltpu.VMEM((rows_per * w,), jnp.float32),
        pltpu.VMEM((rows_per, KP), jnp.float32),
        pltpu.VMEM((w * 16,), jnp.float32),
        pltpu.SemaphoreType.DMA,
        pltpu.SemaphoreType.DMA,
        pltpu.SemaphoreType.DMA,
        pltpu.SemaphoreType.DMA,
    ]

    @functools.partial(
        pl.kernel,
        mesh=mesh,
        compiler_params=pltpu.CompilerParams(needs_layout_passes=False),
        out_type=[
            jax.ShapeDtypeStruct((n_rows * w,), jnp.float32),
            jax.ShapeDtypeStruct((n_rows, KP), jnp.float32),
        ],
        scratch_types=scratch,
    )
    def sc_attend(s_hbm, idx_hbm, attn_hbm, a_hbm, s_v, idx_v, attn_v, a_v,
                  e_buf, sem_e, sem_o, a_sem, at_sem):
        wid = lax.axis_index("s") * nc + lax.axis_index("c")
        rbase = wid * rows_per

        lane = lax.iota(jnp.int32, 16)
        zero16 = jnp.zeros((16,), jnp.float32)
        neg16 = jnp.full((16,), NEG, jnp.float32)
        one16 = jnp.ones((16,), jnp.float32)

        def s_copy(g, sem):
            return pltpu.make_async_copy(
                s_hbm.at[pl.ds(rbase + g * 16, 16)],
                s_v.at[pl.ds(g * 16, 16)], sem)

        def a_copy(g):
            return pltpu.make_async_copy(
                a_v.at[pl.ds(g * 16, 16)],
                a_hbm.at[pl.ds(rbase + g * 16, 16)], a_sem)

        def at_copy(g):
            return pltpu.make_async_copy(
                attn_v.at[pl.ds(g * 16 * w, 16 * w)],
                attn_hbm.at[pl.ds((rbase + g * 16) * w, 16 * w)], at_sem)

        s_copy(0, sem_e).start()
        s_copy(1, sem_o).start()
        pltpu.sync_copy(idx_hbm.at[pl.ds(rbase * w, rows_per * w)], idx_v)

        def group(g, sem):
            def zbody(r, c):
                for u in range(KP // 16):
                    a_v[g * 16 + r, pl.ds(u * 16, 16)] = zero16
                return c

            lax.fori_loop(0, 16, zbody, 0)
            s_copy(g, sem).wait()

            row16 = g * 16 + lane
            idx_off = row16 * w

            def pass1(wi, carry):
                mx, sm = carry
                iw = plsc.load_gather(idx_v, [idx_off + wi])
                e = plsc.load_gather(s_v, [row16, iw])
                e = jnp.where(iw == MASK_ID, neg16, e)
                e_buf[pl.ds(wi * 16, 16)] = e
                nmx = jnp.maximum(mx, e)
                sm = sm * jnp.exp(mx - nmx) + jnp.exp(e - nmx)
                return nmx, sm

            mx, sm = lax.fori_loop(0, w, pass1, (neg16, zero16))
            inv = one16 / sm

            def pass2(wi, carry):
                e = e_buf[pl.ds(wi * 16, 16)]
                aw = jnp.exp(e - mx) * inv
                iw = plsc.load_gather(idx_v, [idx_off + wi])
                plsc.store_scatter(attn_v, [idx_off + wi], aw)
                plsc.addupdate_scatter(a_v, [row16, iw], aw)
                return carry

            lax.fori_loop(0, w, pass2, 0)
            a_copy(g).start()
            at_copy(g).start()

        def tbody(t, c):
            g0 = 2 * t

            group(g0, sem_e)

            @pl.when(t + 1 < n_t)
            def _():
                s_copy(g0 + 2, sem_e).start()

            group(g0 + 1, sem_o)

            @pl.when(t + 1 < n_t)
            def _():
                s_copy(g0 + 3, sem_o).start()

            return c

        lax.fori_loop(0, n_t, tbody, 0)

        def drain(g, c):
            a_copy(g).wait()
            at_copy(g).wait()
            return c

        lax.fori_loop(0, n_g, drain, 0)

    return sc_attend


def kernel(value, key, query, mask_ori, Wv, Wk, Wq, Wf, bf):
    n, l, hd = query.shape
    vd = value.shape[2]
    w = mask_ori.shape[2]
    nl = n * l

    scores = pl.pallas_call(
        _score_body,
        grid=(n,),
        in_specs=[
            pl.BlockSpec((1, l, hd), lambda i: (i, 0, 0)),
            pl.BlockSpec((1, KP, hd), lambda i: (i, 0, 0)),
            pl.BlockSpec((hd, hd), lambda i: (0, 0)),
            pl.BlockSpec((hd, hd), lambda i: (0, 0)),
        ],
        out_specs=pl.BlockSpec((l, KP), lambda i: (i, 0)),
        out_shape=jax.ShapeDtypeStruct((nl, KP), jnp.float32),
    )(query, key, Wq, Wk)

    attn_flat, a_dense = _make_sc_attend(nl, w)(
        scores, mask_ori.reshape(nl * w))

    attn = attn_flat.reshape(n, l, w)

    out = pl.pallas_call(
        _out_body,
        grid=(n,),
        in_specs=[
            pl.BlockSpec((l, KP), lambda i: (i, 0)),
            pl.BlockSpec((1, KP, vd), lambda i: (i, 0, 0)),
            pl.BlockSpec((vd, vd), lambda i: (0, 0)),
            pl.BlockSpec((vd, vd), lambda i: (0, 0)),
            pl.BlockSpec((1, vd), lambda i: (0, 0)),
        ],
        out_specs=pl.BlockSpec((1, l, vd), lambda i: (i, 0, 0)),
        out_shape=jax.ShapeDtypeStruct((n, l, vd), jnp.float32),
    )(a_dense, value, Wv, Wf, bf.reshape(1, vd))

    return out, attn
